# Initial kernel scaffold; baseline (speedup 1.0000x reference)
import jax
import jax.numpy as jnp
from jax.experimental import pallas as pl


def kernel(indices, embeddings):
    raise NotImplementedError("write your pallas kernel here")


# speedup vs baseline: 1.0228x; 1.0228x over previous
import functools

import jax
import jax.numpy as jnp
from jax import lax
from jax.experimental import pallas as pl
from jax.experimental.pallas import tpu as pltpu
from jax.experimental.pallas import tpu_sc as plsc

VOCAB = 1000000
EMBED = 32
BATCH = 16384
HIST = 50

_B = BATCH * HIST
_NW = 32
_BPW = _B // _NW
_CH = 128
_NCH = _BPW // _CH

_mesh = plsc.VectorSubcoreMesh(core_axis_name="c", subcore_axis_name="s")


@functools.partial(
    pl.kernel,
    mesh=_mesh,
    out_type=jax.ShapeDtypeStruct((_B, EMBED), jnp.float32),
    scratch_types=[
        pltpu.VMEM((_NCH, _CH), jnp.int32),
        pltpu.VMEM((_CH, EMBED), jnp.float32),
        pltpu.VMEM((_CH, EMBED), jnp.float32),
        pltpu.SemaphoreType.DMA,
        pltpu.SemaphoreType.DMA,
    ],
    compiler_params=pltpu.CompilerParams(use_tc_tiling_on_sc=False),
)
def _emb_lookup(idx_hbm, table_hbm, out_hbm, idx_v, rows0, rows1, sem0, sem1):
    wid = lax.axis_index("s") * 2 + lax.axis_index("c")
    base = wid * _BPW
    pltpu.sync_copy(idx_hbm.at[wid], idx_v)

    def body(j, _):
        cp = pltpu.async_copy(table_hbm.at[idx_v.at[j]], rows0, sem0)
        cp.wait()
        pltpu.sync_copy(rows0, out_hbm.at[pl.ds(base + j * _CH, _CH)])
        return _

    lax.fori_loop(0, _NCH, body, None)


def kernel(indices, embeddings):
    idx = indices.reshape(_NW, _NCH, _CH).astype(jnp.int32)
    out = _emb_lookup(idx, embeddings)
    return out.reshape(BATCH, HIST, EMBED)


# speedup vs baseline: 1.1092x; 1.0845x over previous
import functools

import jax
import jax.numpy as jnp
from jax import lax
from jax.experimental import pallas as pl
from jax.experimental.pallas import tpu as pltpu
from jax.experimental.pallas import tpu_sc as plsc

VOCAB = 1000000
EMBED = 32
BATCH = 16384
HIST = 50

_B = BATCH * HIST
_NW = 32
_BPW = _B // _NW
_CH = 128
_NCH = _BPW // _CH
_D = 4

_mesh = plsc.VectorSubcoreMesh(core_axis_name="c", subcore_axis_name="s")


@functools.partial(
    pl.kernel,
    mesh=_mesh,
    out_type=jax.ShapeDtypeStruct((_B, EMBED), jnp.float32),
    scratch_types=[
        pltpu.VMEM((_NCH, _CH), jnp.int32),
        [pltpu.VMEM((_CH, EMBED), jnp.float32) for _ in range(_D)],
        [pltpu.SemaphoreType.DMA for _ in range(_D)],
    ],
    compiler_params=pltpu.CompilerParams(use_tc_tiling_on_sc=False),
)
def _emb_lookup(idx_hbm, table_hbm, out_hbm, idx_v, rows, sems):
    wid = lax.axis_index("s") * 2 + lax.axis_index("c")
    base = wid * _BPW
    pltpu.sync_copy(idx_hbm.at[wid], idx_v)

    for b in range(_D):
        pltpu.async_copy(table_hbm.at[idx_v.at[b]], rows[b], sems[b])

    def outer(g, carry):
        for b in range(_D):
            j = g * _D + b
            pltpu.make_async_copy(table_hbm.at[pl.ds(0, _CH)], rows[b], sems[b]).wait()
            pltpu.sync_copy(rows[b], out_hbm.at[pl.ds(base + j * _CH, _CH)])
            pltpu.async_copy(table_hbm.at[idx_v.at[j + _D]], rows[b], sems[b])
        return carry

    lax.fori_loop(0, _NCH // _D - 1, outer, None)

    for b in range(_D):
        j = _NCH - _D + b
        pltpu.make_async_copy(table_hbm.at[pl.ds(0, _CH)], rows[b], sems[b]).wait()
        pltpu.sync_copy(rows[b], out_hbm.at[pl.ds(base + j * _CH, _CH)])


def kernel(indices, embeddings):
    idx = indices.reshape(_NW, _NCH, _CH).astype(jnp.int32)
    out = _emb_lookup(idx, embeddings)
    return out.reshape(BATCH, HIST, EMBED)


# speedup vs baseline: 1.1126x; 1.0030x over previous
import functools

import jax
import jax.numpy as jnp
from jax import lax
from jax.experimental import pallas as pl
from jax.experimental.pallas import tpu as pltpu
from jax.experimental.pallas import tpu_sc as plsc

VOCAB = 1000000
EMBED = 32
BATCH = 16384
HIST = 50

_B = BATCH * HIST
_NW = 32
_BPW = _B // _NW
_CH = 128
_SUP = 10
_SC_ROWS = _CH * _SUP
_NS = _BPW // _SC_ROWS
_NCH = _BPW // _CH

_mesh = plsc.VectorSubcoreMesh(core_axis_name="c", subcore_axis_name="s")


@functools.partial(
    pl.kernel,
    mesh=_mesh,
    out_type=jax.ShapeDtypeStruct((_B, EMBED), jnp.float32),
    scratch_types=[
        pltpu.VMEM((_NCH, _CH), jnp.int32),
        [pltpu.VMEM((_SC_ROWS, EMBED), jnp.float32) for _ in range(2)],
        [pltpu.SemaphoreType.DMA for _ in range(2)],
        [pltpu.SemaphoreType.DMA for _ in range(2)],
    ],
    compiler_params=pltpu.CompilerParams(use_tc_tiling_on_sc=False),
)
def _emb_lookup(idx_hbm, table_hbm, out_hbm, idx_v, rows, gsem, wsem):
    wid = lax.axis_index("s") * 2 + lax.axis_index("c")
    base = wid * _BPW
    pltpu.sync_copy(idx_hbm.at[wid], idx_v)

    def issue(s, buf):
        for b in range(_SUP):
            pltpu.async_copy(
                table_hbm.at[idx_v.at[s * _SUP + b]],
                rows[buf].at[pl.ds(b * _CH, _CH)],
                gsem[buf],
            )

    def drain_gathers(buf):
        for b in range(_SUP):
            pltpu.make_async_copy(
                table_hbm.at[pl.ds(0, _CH)],
                rows[buf].at[pl.ds(b * _CH, _CH)],
                gsem[buf],
            ).wait()

    def wait_writeout(buf):
        pltpu.make_async_copy(
            table_hbm.at[pl.ds(0, _SC_ROWS)], rows[buf], wsem[buf]
        ).wait()

    issue(0, 0)
    issue(1, 1)

    def outer(g, carry):
        s0 = g * 2
        for p in range(2):
            s = s0 + p
            buf = p
            drain_gathers(buf)
            pltpu.async_copy(
                rows[buf],
                out_hbm.at[pl.ds(base + s * _SC_ROWS, _SC_ROWS)],
                wsem[buf],
            )
            wait_writeout(buf)
            issue(s + 2, buf)
        return carry

    lax.fori_loop(0, _NS // 2 - 1, outer, None)

    for p in range(2):
        s = _NS - 2 + p
        drain_gathers(p)
        pltpu.async_copy(
            rows[p], out_hbm.at[pl.ds(base + s * _SC_ROWS, _SC_ROWS)], wsem[p]
        )
    for p in range(2):
        wait_writeout(p)


def kernel(indices, embeddings):
    idx = indices.reshape(_NW, _NCH, _CH).astype(jnp.int32)
    out = _emb_lookup(idx, embeddings)
    return out.reshape(BATCH, HIST, EMBED)


# speedup vs baseline: 1.8294x; 1.6443x over previous
import functools

import jax
import jax.numpy as jnp
from jax import lax
from jax.experimental import pallas as pl
from jax.experimental.pallas import tpu as pltpu
from jax.experimental.pallas import tpu_sc as plsc

VOCAB = 1000000
EMBED = 32
BATCH = 16384
HIST = 50

_NW = 32
_CH = 128
_NBT = BATCH // _CH
_NB = HIST * _NBT
_BPW = _NB // _NW
_TB = EMBED * _CH

_mesh = plsc.VectorSubcoreMesh(core_axis_name="c", subcore_axis_name="s")


@functools.partial(
    pl.kernel,
    mesh=_mesh,
    out_type=jax.ShapeDtypeStruct((HIST, EMBED // 8, _NBT, 8 * _CH), jnp.float32),
    scratch_types=[
        pltpu.VMEM((_BPW, _CH), jnp.int32),
        [pltpu.VMEM((_CH, EMBED), jnp.float32) for _ in range(2)],
        [pltpu.VMEM((_TB,), jnp.float32) for _ in range(2)],
        [pltpu.SemaphoreType.DMA for _ in range(2)],
        [pltpu.SemaphoreType.DMA for _ in range(2)],
    ],
    compiler_params=pltpu.CompilerParams(
        use_tc_tiling_on_sc=False, needs_layout_passes=False
    ),
)
def _emb_lookup(idx_hbm, table_hbm, out_hbm, idx_v, rows, tbuf, gsem, wsem):
    wid = lax.axis_index("s") * 2 + lax.axis_index("c")
    base = wid * _BPW
    pltpu.sync_copy(idx_hbm.at[pl.ds(base, _BPW)], idx_v)

    lane_step = lax.iota(jnp.int32, 16) * _CH

    def issue_gather(k, buf):
        pltpu.async_copy(table_hbm.at[idx_v.at[k]], rows[buf], gsem[buf])

    def wait_gather(buf):
        pltpu.make_async_copy(table_hbm.at[pl.ds(0, _CH)], rows[buf], gsem[buf]).wait()

    def transpose(buf):
        for l in range(_CH):
            for half in range(2):
                v = rows[buf][l, pl.ds(half * 16, 16)]
                addr = lane_step + (half * 16 * _CH + l)
                plsc.store_scatter(tbuf[buf], [addr], v)

    def issue_write(k, buf):
        blk = base + k
        h = blk >> 7
        bt = blk & 127
        for et in range(EMBED // 8):
            pltpu.async_copy(
                tbuf[buf].at[pl.ds(et * 1024, 1024)],
                out_hbm.at[h, et, bt],
                wsem[buf],
            )

    def wait_write(buf):
        for et in range(EMBED // 8):
            pltpu.make_async_copy(
                out_hbm.at[0, 0, 0],
                tbuf[buf].at[pl.ds(et * 1024, 1024)],
                wsem[buf],
            ).wait()

    issue_gather(0, 0)
    issue_gather(1, 1)
    for buf in range(2):
        wait_gather(buf)
        transpose(buf)
        issue_write(buf, buf)
        issue_gather(buf + 2, buf)

    def outer(g, carry):
        for p in range(2):
            k = g * 2 + p
            wait_gather(p)
            wait_write(p)
            transpose(p)
            issue_write(k, p)
            issue_gather(k + 2, p)
        return carry

    lax.fori_loop(1, _BPW // 2 - 1, outer, None)

    for p in range(2):
        k = _BPW - 2 + p
        wait_gather(p)
        wait_write(p)
        transpose(p)
        issue_write(k, p)
    for p in range(2):
        wait_write(p)


def kernel(indices, embeddings):
    idx = indices.astype(jnp.int32).T.reshape(_NB, _CH)
    out4d = _emb_lookup(idx, embeddings)
    out5d = out4d.reshape(HIST, EMBED // 8, _NBT, 8, _CH)
    return out5d.transpose(2, 4, 0, 1, 3).reshape(BATCH, HIST, EMBED)


# speedup vs baseline: 2.2400x; 1.2245x over previous
import functools

import jax
import jax.numpy as jnp
from jax import lax
from jax.experimental import pallas as pl
from jax.experimental.pallas import tpu as pltpu
from jax.experimental.pallas import tpu_sc as plsc

VOCAB = 1000000
EMBED = 32
BATCH = 16384
HIST = 50

_NW = 32
_CH = 128
_NBT = BATCH // _CH
_NB = HIST * _NBT
_BPW = _NB // _NW
_TB = EMBED * _CH
_TSKEW = _CH + 8

_mesh = plsc.VectorSubcoreMesh(core_axis_name="c", subcore_axis_name="s")


@functools.partial(
    pl.kernel,
    mesh=_mesh,
    out_type=jax.ShapeDtypeStruct((HIST, EMBED // 8, _NBT, 8, _CH), jnp.float32),
    scratch_types=[
        pltpu.VMEM((_BPW, _CH), jnp.int32),
        [pltpu.VMEM((_CH, EMBED), jnp.float32) for _ in range(2)],
        [pltpu.VMEM((_TSKEW * EMBED,), jnp.float32) for _ in range(2)],
        [pltpu.SemaphoreType.DMA for _ in range(2)],
        [pltpu.SemaphoreType.DMA for _ in range(2)],
    ],
    compiler_params=pltpu.CompilerParams(
        use_tc_tiling_on_sc=False, needs_layout_passes=False
    ),
)
def _emb_lookup(idx_hbm, table_hbm, out_hbm, idx_v, rows, tbuf, gsem, wsem):
    wid = lax.axis_index("s") * 2 + lax.axis_index("c")
    base = wid * _BPW
    pltpu.sync_copy(idx_hbm.at[pl.ds(base, _BPW)], idx_v)

    lane_step = lax.iota(jnp.int32, 16) * _TSKEW

    def issue_gather(k, buf):
        pltpu.async_copy(table_hbm.at[idx_v.at[k]], rows[buf], gsem[buf])

    def wait_gather(buf):
        pltpu.make_async_copy(table_hbm.at[pl.ds(0, _CH)], rows[buf], gsem[buf]).wait()

    def transpose(buf):
        for l in range(_CH):
            for half in range(2):
                v = rows[buf][l, pl.ds(half * 16, 16)]
                addr = lane_step + (half * 16 * _TSKEW + l)
                plsc.store_scatter(tbuf[buf], [addr], v)

    def issue_write(k, buf):
        blk = base + k
        h = blk >> 7
        bt = blk & 127
        for e in range(EMBED):
            pltpu.async_copy(
                tbuf[buf].at[pl.ds(e * _TSKEW, _CH)],
                out_hbm.at[h, e // 8, bt, e % 8],
                wsem[buf],
            )

    def wait_write(buf):
        for e in range(EMBED):
            pltpu.make_async_copy(
                out_hbm.at[0, 0, 0, 0],
                tbuf[buf].at[pl.ds(e * _TSKEW, _CH)],
                wsem[buf],
            ).wait()

    issue_gather(0, 0)
    issue_gather(1, 1)
    for buf in range(2):
        wait_gather(buf)
        transpose(buf)
        issue_write(buf, buf)
        issue_gather(buf + 2, buf)

    def outer(g, carry):
        for p in range(2):
            k = g * 2 + p
            wait_gather(p)
            wait_write(p)
            transpose(p)
            issue_write(k, p)
            issue_gather(k + 2, p)
        return carry

    lax.fori_loop(1, _BPW // 2 - 1, outer, None)

    for p in range(2):
        k = _BPW - 2 + p
        wait_gather(p)
        wait_write(p)
        transpose(p)
        issue_write(k, p)
    for p in range(2):
        wait_write(p)


def kernel(indices, embeddings):
    idx = indices.astype(jnp.int32).T.reshape(_NB, _CH)
    out5d = _emb_lookup(idx, embeddings)
    return out5d.transpose(2, 4, 0, 1, 3).reshape(BATCH, HIST, EMBED)
